# Initial kernel scaffold; baseline (speedup 1.0000x reference)
import functools

import jax
import jax.numpy as jnp
from jax import lax
from jax.experimental import pallas as pl
from jax.experimental.pallas import tpu as pltpu
from jax.experimental.pallas import tpu_sc as plsc

N = 10000
E = 160000
T = 320000
H = 128
EMB = 64
RBE = 80
RBA = 40
NZ = 108
NA = 4
NG = 4
CUT = 5.0

NCORE = 2
NSUB = 16
NWK = NCORE * NSUB
CH = 128
NP = 10112

f32 = jnp.float32
i32 = jnp.int32


def _mesh():
    return plsc.VectorSubcoreMesh(core_axis_name="c", subcore_axis_name="s",
                                  num_cores=NCORE, num_subcores=NSUB)


def _ln(h):
    mu = jnp.mean(h, axis=-1, keepdims=True)
    var = jnp.mean((h - mu) ** 2, axis=-1, keepdims=True)
    return (h - mu) / jnp.sqrt(var + 1e-5)


def _silu(x):
    return x * jax.nn.sigmoid(x)


def make_gather_abh(S, R):
    NCHK = R // CH
    per = -(-NCHK // NWK)

    @functools.partial(
        pl.kernel,
        out_type=[jax.ShapeDtypeStruct((R, H), f32)] * 3,
        mesh=_mesh(),
        scratch_types=[
            pltpu.VMEM((CH,), i32), pltpu.VMEM((CH,), i32),
            pltpu.VMEM((CH, H), f32), pltpu.VMEM((CH, H), f32),
            pltpu.VMEM((CH, H), f32),
            pltpu.SemaphoreType.DMA, pltpu.SemaphoreType.DMA,
            pltpu.SemaphoreType.DMA,
        ],
    )
    def k(ta, tb, th, idx2, oa, ob, oh, ivs, ivd, ra, rb, rh, s0, s1, s2):
        wid = lax.axis_index("s") * NCORE + lax.axis_index("c")

        def body(j, carry):
            chk = j * NWK + wid

            @pl.when(chk < NCHK)
            def _():
                base = chk * CH
                pltpu.sync_copy(idx2.at[0, pl.ds(base, CH)], ivs)
                pltpu.sync_copy(idx2.at[1, pl.ds(base, CH)], ivd)
                ca = pltpu.async_copy(ta.at[ivs], ra, s0)
                cb = pltpu.async_copy(tb.at[ivd], rb, s1)
                ch_ = pltpu.async_copy(th.at[ivd], rh, s2)
                ca.wait()
                cb.wait()
                ch_.wait()
                pltpu.sync_copy(ra, oa.at[pl.ds(base, CH)])
                pltpu.sync_copy(rb, ob.at[pl.ds(base, CH)])
                pltpu.sync_copy(rh, oh.at[pl.ds(base, CH)])

            return carry

        lax.fori_loop(0, per, body, 0)

    return k


def make_gather_pair(V, R, D):
    NCHK = R // CH
    per = -(-NCHK // NWK)

    @functools.partial(
        pl.kernel,
        out_type=[jax.ShapeDtypeStruct((R, D), f32)] * 2,
        mesh=_mesh(),
        scratch_types=[
            pltpu.VMEM((CH,), i32), pltpu.VMEM((CH,), i32),
            pltpu.VMEM((CH, D), f32), pltpu.VMEM((CH, D), f32),
            pltpu.SemaphoreType.DMA, pltpu.SemaphoreType.DMA,
        ],
    )
    def k(tab, idx2, oa, ob, ivs, ivd, ra, rb, s0, s1):
        wid = lax.axis_index("s") * NCORE + lax.axis_index("c")

        def body(j, carry):
            chk = j * NWK + wid

            @pl.when(chk < NCHK)
            def _():
                base = chk * CH
                pltpu.sync_copy(idx2.at[0, pl.ds(base, CH)], ivs)
                pltpu.sync_copy(idx2.at[1, pl.ds(base, CH)], ivd)
                ca = pltpu.async_copy(tab.at[ivs], ra, s0)
                cb = pltpu.async_copy(tab.at[ivd], rb, s1)
                ca.wait()
                cb.wait()
                pltpu.sync_copy(ra, oa.at[pl.ds(base, CH)])
                pltpu.sync_copy(rb, ob.at[pl.ds(base, CH)])

            return carry

        lax.fori_loop(0, per, body, 0)

    return k


def make_gather_emb(V, R):
    NCHK = R // CH
    per = -(-NCHK // NWK)

    @functools.partial(
        pl.kernel,
        out_type=jax.ShapeDtypeStruct((R, H), f32),
        mesh=_mesh(),
        scratch_types=[
            pltpu.VMEM((CH,), i32), pltpu.VMEM((CH, H), f32),
            pltpu.SemaphoreType.DMA,
        ],
    )
    def k(tab, idx, out, iv, rv, s0):
        wid = lax.axis_index("s") * NCORE + lax.axis_index("c")

        def body(j, carry):
            chk = j * NWK + wid

            @pl.when(chk < NCHK)
            def _():
                base = chk * CH
                pltpu.sync_copy(idx.at[pl.ds(base, CH)], iv)
                pltpu.async_copy(tab.at[iv], rv, s0).wait()
                pltpu.sync_copy(rv, out.at[pl.ds(base, CH)])

            return carry

        lax.fori_loop(0, per, body, 0)

    return k


def make_scatter(S, R, WS, NWIN):
    WSP = WS + 16
    SW = WSP // 16
    ZR = 64
    assert WSP % 16 == 0 and SW >= ZR and WS % 16 == 0 and S % 16 == 0

    @functools.partial(
        pl.kernel,
        out_type=jax.ShapeDtypeStruct((2, S, H), f32),
        mesh=_mesh(),
        scratch_types=[
            pltpu.VMEM((CH,), i32),
            pltpu.VMEM((CH,), i32),
            pltpu.VMEM((CH, H), f32),
            pltpu.VMEM((ZR, H), f32),
            pltpu.VMEM((32,), i32),
            pltpu.VMEM_SHARED((WSP, H), f32),
        ],
    )
    def k(vals, keys, offs, out, kv, lv, rv, zb, ov, sh):
        cc = lax.axis_index("c")
        sid = lax.axis_index("s")
        lane = lax.broadcasted_iota(i32, (16,), 0)

        def zb_body(t, carry):
            zb[t // 8, pl.ds((t % 8) * 16, 16)] = jnp.zeros((16,), f32)
            return carry

        lax.fori_loop(0, ZR * 8, zb_body, 0)

        for w in range(NWIN):
            vw = min(WS, S - w * WS)
            ow = vw // 16

            def z_body(t, carry):
                off = jnp.minimum(t * ZR, SW - ZR)
                pltpu.sync_copy(zb, sh.at[pl.ds(sid * SW + off, ZR)])
                return carry

            lax.fori_loop(0, -(-SW // ZR), z_body, 0)
            plsc.subcore_barrier()

            pltpu.sync_copy(offs.at[w], ov)
            v0 = ov[pl.ds(0, 16)]
            v1 = ov[pl.ds(16, 16)]
            t0 = jnp.max(jnp.where(lane == sid, v0, jnp.int32(-2**31)))
            t1 = jnp.max(jnp.where(lane == sid, v1, jnp.int32(-2**31)))

            def c_body(ci, carry):
                base = ci * CH
                pltpu.sync_copy(keys.at[pl.ds(base, CH)], kv)
                for q in range(CH // 16):
                    kk = kv[pl.ds(q * 16, 16)]
                    li = kk - jnp.int32(w * WS)
                    ok = (li >= 0) & (li < WS)
                    lv[pl.ds(q * 16, 16)] = jnp.where(ok, li, jnp.int32(WS))
                pltpu.sync_copy(vals.at[cc, pl.ds(base, CH)], rv)
                pltpu.sync_copy(rv, sh.at[lv], add=True)
                return carry

            lax.fori_loop(t0, t1, c_body, 0)
            plsc.subcore_barrier()

            pltpu.sync_copy(sh.at[pl.ds(sid * ow, ow)],
                            out.at[cc, pl.ds(w * WS + sid * ow, ow)])
            plsc.subcore_barrier()

    return k


def _full(shape):
    return pl.BlockSpec(shape, lambda i: tuple(0 for _ in shape))


def make_bonds(blk):
    grid = (E // blk,)
    centers = jnp.linspace(0.0, 8.0, RBE)
    gamma = 1.0 / (8.0 / (RBE - 1)) ** 2

    def body(r_ref, w1, b1, w2, b2, y_ref, rt_ref):
        r = r_ref[...]
        bl = jnp.sqrt(jnp.sum(r * r, axis=1, keepdims=True))
        rbf = jnp.exp(-gamma * (bl - centers[None, :]) ** 2)
        h1 = _silu(_ln(jnp.dot(rbf, w1[...], preferred_element_type=f32)
                       + b1[...]))
        y = _silu(_ln(jnp.dot(h1, w2[...], preferred_element_type=f32)
                      + b2[...]))
        y_ref[...] = y
        fc2 = jnp.where(bl < CUT, 0.5 * (jnp.cos(jnp.pi * bl / CUT) + 1.0),
                        0.0)
        rt_ref[...] = jnp.concatenate(
            [r, fc2, jnp.zeros((blk, 12), f32)], axis=1)

    return pl.pallas_call(
        body,
        grid=grid,
        in_specs=[pl.BlockSpec((blk, 3), lambda i: (i, 0)),
                  _full((RBE, EMB)), _full((1, EMB)),
                  _full((EMB, H)), _full((1, H))],
        out_specs=[pl.BlockSpec((blk, H), lambda i: (i, 0)),
                   pl.BlockSpec((blk, 16), lambda i: (i, 0))],
        out_shape=[jax.ShapeDtypeStruct((E, H), f32),
                   jax.ShapeDtypeStruct((E, 16), f32)],
    )


def make_angles(blk):
    grid = (T // blk,)
    centers = jnp.linspace(-1.0, 1.0, RBA)
    gamma = 1.0 / (2.0 / (RBA - 1)) ** 2

    def body(ra_ref, rb_ref, w1, b1, w2, b2, z_ref):
        ra = ra_ref[...]
        rb = rb_ref[...]
        r1 = -ra[:, 0:3]
        r2 = rb[:, 0:3]
        dot = jnp.sum(r1 * r2, axis=1, keepdims=True)
        n1 = jnp.sqrt(jnp.sum(r1 * r1, axis=1, keepdims=True))
        n2 = jnp.sqrt(jnp.sum(r2 * r2, axis=1, keepdims=True))
        cosang = jnp.clip(dot / (n1 * n2 + 1e-12), -1.0, 1.0)
        rbf = jnp.exp(-gamma * (cosang - centers[None, :]) ** 2)
        h1 = _silu(_ln(jnp.dot(rbf, w1[...], preferred_element_type=f32)
                       + b1[...]))
        z = _silu(_ln(jnp.dot(h1, w2[...], preferred_element_type=f32)
                      + b2[...]))
        fcut3 = ra[:, 3:4] * rb[:, 3:4]
        z_ref[...] = z * fcut3

    return pl.pallas_call(
        body,
        grid=grid,
        in_specs=[pl.BlockSpec((blk, 16), lambda i: (i, 0)),
                  pl.BlockSpec((blk, 16), lambda i: (i, 0)),
                  _full((RBA, EMB)), _full((1, EMB)),
                  _full((EMB, H)), _full((1, H))],
        out_specs=pl.BlockSpec((blk, H), lambda i: (i, 0)),
        out_shape=jax.ShapeDtypeStruct((T, H), f32),
    )


def make_proj3(S, blk):
    grid = (S // blk,)

    def body(x_ref, w_ref, b_ref, a_ref, b2_ref, h_ref):
        x = x_ref[...]
        a_ref[...] = jnp.dot(x, w_ref[0], preferred_element_type=f32) + b_ref[0]
        b2_ref[...] = jnp.dot(x, w_ref[1], preferred_element_type=f32) + b_ref[1]
        h_ref[...] = jnp.dot(x, w_ref[2], preferred_element_type=f32) + b_ref[2]

    return pl.pallas_call(
        body,
        grid=grid,
        in_specs=[pl.BlockSpec((blk, H), lambda i: (i, 0)),
                  pl.BlockSpec((3, H, H), lambda i: (0, 0, 0)),
                  pl.BlockSpec((3, 1, H), lambda i: (0, 0, 0))],
        out_specs=[pl.BlockSpec((blk, H), lambda i: (i, 0))] * 3,
        out_shape=[jax.ShapeDtypeStruct((S, H), f32)] * 3,
    )


def make_gate(R, blk, skip_edgenorm):
    grid = (R // blk,)

    def body(ga_ref, gb_ref, gh_ref, y_ref, w2, b2, nd_ref, yn_ref):
        y = y_ref[...]
        m = (ga_ref[...] + gb_ref[...]
             + jnp.dot(y, w2[...], preferred_element_type=f32) + b2[...])
        sig = jax.nn.sigmoid(m)
        num = sig * gh_ref[...]
        nd_ref[...] = jnp.stack([num, sig], axis=0)
        ym = m if skip_edgenorm else _ln(m)
        yn_ref[...] = y + _silu(ym)

    return pl.pallas_call(
        body,
        grid=grid,
        in_specs=[pl.BlockSpec((blk, H), lambda i: (i, 0))] * 4
                 + [_full((H, H)), _full((1, H))],
        out_specs=[pl.BlockSpec((2, blk, H), lambda i: (0, i, 0)),
                   pl.BlockSpec((blk, H), lambda i: (i, 0))],
        out_shape=[jax.ShapeDtypeStruct((2, R, H), f32),
                   jax.ShapeDtypeStruct((R, H), f32)],
    )


def make_update(S, blk):
    grid = (S // blk,)

    def body(x_ref, nd_ref, w4, b4, o_ref):
        x = x_ref[...]
        h = nd_ref[0] / (nd_ref[1] + 1e-6)
        o_ref[...] = x + _silu(_ln(
            jnp.dot(x, w4[...], preferred_element_type=f32) + b4[...] + h))

    return pl.pallas_call(
        body,
        grid=grid,
        in_specs=[pl.BlockSpec((blk, H), lambda i: (i, 0)),
                  pl.BlockSpec((2, blk, H), lambda i: (0, i, 0)),
                  _full((H, H)), _full((1, H))],
        out_specs=pl.BlockSpec((blk, H), lambda i: (i, 0)),
        out_shape=jax.ShapeDtypeStruct((S, H), f32),
    )


def make_readout():
    def body(x_ref, w_ref, b_ref, o_ref):
        s = jnp.sum(x_ref[...], axis=0, keepdims=True)
        o_ref[...] = (jnp.sum(s * w_ref[...], keepdims=True).reshape(1, 1)
                      + b_ref[...] * N)

    return pl.pallas_call(
        body,
        in_specs=[pl.BlockSpec((N, H), lambda: (0, 0)),
                  pl.BlockSpec((1, H), lambda: (0, 0)),
                  pl.BlockSpec((1, 1), lambda: (0, 0))],
        out_specs=pl.BlockSpec((1, 1), lambda: (0, 0)),
        out_shape=jax.ShapeDtypeStruct((1, 1), f32),
    )


WS_N = N
WS_E = 16000
NWIN_E = E // WS_E


def _tile_offsets(woff):
    a = woff[:-1] // CH
    b = -(-woff[1:] // CH)
    sgrid = jnp.arange(17, dtype=i32)
    tt = a[:, None] + ((b - a)[:, None] * sgrid[None, :]) // 16
    return jnp.concatenate([tt[:, :16], tt[:, 1:17]], axis=1).astype(i32)


def kernel(r, atomic_number, edge_index, lg_edge_index, atom_emb,
           edge_W1, edge_b1, edge_W2, edge_b2,
           angle_W1, angle_b1, angle_W2, angle_b2,
           alignn_W, alignn_b, gcn_W, gcn_b, fc_W, fc_b):
    edge_index = edge_index.astype(i32)
    lg = lg_edge_index.astype(i32)
    src = edge_index[0]

    perm = jnp.argsort(lg[0])
    lsrc = lg[0][perm]
    ldst = lg[1][perm]
    lidx2 = jnp.stack([lsrc, ldst])

    woff_n = jnp.array([0, E], dtype=i32)
    offs_n = _tile_offsets(woff_n)
    woff_e = jnp.searchsorted(lsrc, (jnp.arange(NWIN_E + 1) * WS_E)
                              .astype(i32)).astype(i32)
    offs_e = _tile_offsets(woff_e)

    g_emb = make_gather_emb(NZ, NP)
    g_abh_n = make_gather_abh(N, E)
    g_abh_e = make_gather_abh(E, T)
    g_rt = make_gather_pair(E, T, 16)
    sc_n = make_scatter(N, E, WS_N, 1)
    sc_e = make_scatter(E, T, WS_E, NWIN_E)

    bonds = make_bonds(2000)
    angles = make_angles(2000)
    proj3_n = make_proj3(N, 2000)
    proj3_e = make_proj3(E, 2000)
    gate_n = {s: make_gate(E, 2000, s) for s in (False, True)}
    gate_e = {s: make_gate(T, 2000, s) for s in (False, True)}
    upd_n = make_update(N, 2000)
    upd_e = make_update(E, 2000)
    readout = make_readout()

    an_pad = jnp.pad(atomic_number.astype(i32), (0, NP - N))
    x = g_emb(atom_emb, an_pad)[:N]

    y, rtab = bonds(r, edge_W1, edge_b1.reshape(1, EMB),
                    edge_W2, edge_b2.reshape(1, H))

    ra, rb = g_rt(rtab, lidx2)
    z = angles(ra, rb, angle_W1, angle_b1.reshape(1, EMB),
               angle_W2, angle_b2.reshape(1, H))

    def conv_n(x, y, W, b, skip):
        w013 = jnp.stack([W[0], W[1], W[3]])
        b013 = jnp.stack([b[0], b[1], b[3]]).reshape(3, 1, H)
        a, bb, hh = proj3_n(x, w013, b013)
        ga, gb, gh = g_abh_n(a, bb, hh, edge_index)
        nd, ynew = gate_n[skip](ga, gb, gh, y, W[2], b[2].reshape(1, H))
        acc = sc_n(nd, src, offs_n)
        xnew = upd_n(x, acc, W[4], b[4].reshape(1, H))
        return xnew, ynew

    def conv_e(x, y, W, b, skip):
        w013 = jnp.stack([W[0], W[1], W[3]])
        b013 = jnp.stack([b[0], b[1], b[3]]).reshape(3, 1, H)
        a, bb, hh = proj3_e(x, w013, b013)
        ga, gb, gh = g_abh_e(a, bb, hh, lidx2)
        nd, ynew = gate_e[skip](ga, gb, gh, y, W[2], b[2].reshape(1, H))
        acc = sc_e(nd, lsrc, offs_e)
        xnew = upd_e(x, acc, W[4], b[4].reshape(1, H))
        return xnew, ynew

    for i in range(NA):
        skip = (i == NA - 1)
        x, m = conv_n(x, y, alignn_W[i, 0], alignn_b[i, 0], False)
        y, z = conv_e(m, z, alignn_W[i, 1], alignn_b[i, 1], skip)
    for i in range(NG):
        skip = (i == NG - 1)
        x, y = conv_n(x, y, gcn_W[i], gcn_b[i], skip)

    energy = readout(x, fc_W.reshape(1, H), fc_b.reshape(1, 1))
    return jnp.reshape(energy, ())


# speedup vs baseline: 2.6695x; 2.6695x over previous
import functools

import jax
import jax.numpy as jnp
from jax import lax
from jax.experimental import pallas as pl
from jax.experimental.pallas import tpu as pltpu
from jax.experimental.pallas import tpu_sc as plsc

N = 10000
E = 160000
T = 320000
H = 128
EMB = 64
RBE = 80
RBA = 40
NZ = 108
NA = 4
NG = 4
CUT = 5.0

NCORE = 2
NSUB = 16
NWK = NCORE * NSUB
CH = 128
NP = 10112

f32 = jnp.float32
i32 = jnp.int32


def _mesh():
    return plsc.VectorSubcoreMesh(core_axis_name="c", subcore_axis_name="s",
                                  num_cores=NCORE, num_subcores=NSUB)


def _ln(h):
    mu = jnp.mean(h, axis=-1, keepdims=True)
    var = jnp.mean((h - mu) ** 2, axis=-1, keepdims=True)
    return (h - mu) / jnp.sqrt(var + 1e-5)


def _silu(x):
    return x * jax.nn.sigmoid(x)


def make_gather_abh(S, R):
    NCHK = R // CH
    per = -(-NCHK // NWK)

    @functools.partial(
        pl.kernel,
        out_type=[jax.ShapeDtypeStruct((R, H), f32)] * 3,
        mesh=_mesh(),
        scratch_types=[
            pltpu.VMEM((CH,), i32), pltpu.VMEM((CH,), i32),
            pltpu.VMEM((CH, H), f32), pltpu.VMEM((CH, H), f32),
            pltpu.VMEM((CH, H), f32),
            pltpu.SemaphoreType.DMA, pltpu.SemaphoreType.DMA,
            pltpu.SemaphoreType.DMA,
        ],
    )
    def k(ta, tb, th, idx2, oa, ob, oh, ivs, ivd, ra, rb, rh, s0, s1, s2):
        wid = lax.axis_index("s") * NCORE + lax.axis_index("c")

        def body(j, carry):
            chk = j * NWK + wid

            @pl.when(chk < NCHK)
            def _():
                base = chk * CH
                pltpu.sync_copy(idx2.at[0, pl.ds(base, CH)], ivs)
                pltpu.sync_copy(idx2.at[1, pl.ds(base, CH)], ivd)
                ca = pltpu.async_copy(ta.at[ivs], ra, s0)
                cb = pltpu.async_copy(tb.at[ivd], rb, s1)
                ch_ = pltpu.async_copy(th.at[ivd], rh, s2)
                ca.wait()
                cb.wait()
                ch_.wait()
                pltpu.sync_copy(ra, oa.at[pl.ds(base, CH)])
                pltpu.sync_copy(rb, ob.at[pl.ds(base, CH)])
                pltpu.sync_copy(rh, oh.at[pl.ds(base, CH)])

            return carry

        lax.fori_loop(0, per, body, 0)

    return k


def make_gather_pair(V, R, D):
    NCHK = R // CH
    per = -(-NCHK // NWK)

    @functools.partial(
        pl.kernel,
        out_type=[jax.ShapeDtypeStruct((R, D), f32)] * 2,
        mesh=_mesh(),
        scratch_types=[
            pltpu.VMEM((CH,), i32), pltpu.VMEM((CH,), i32),
            pltpu.VMEM((CH, D), f32), pltpu.VMEM((CH, D), f32),
            pltpu.SemaphoreType.DMA, pltpu.SemaphoreType.DMA,
        ],
    )
    def k(tab, idx2, oa, ob, ivs, ivd, ra, rb, s0, s1):
        wid = lax.axis_index("s") * NCORE + lax.axis_index("c")

        def body(j, carry):
            chk = j * NWK + wid

            @pl.when(chk < NCHK)
            def _():
                base = chk * CH
                pltpu.sync_copy(idx2.at[0, pl.ds(base, CH)], ivs)
                pltpu.sync_copy(idx2.at[1, pl.ds(base, CH)], ivd)
                ca = pltpu.async_copy(tab.at[ivs], ra, s0)
                cb = pltpu.async_copy(tab.at[ivd], rb, s1)
                ca.wait()
                cb.wait()
                pltpu.sync_copy(ra, oa.at[pl.ds(base, CH)])
                pltpu.sync_copy(rb, ob.at[pl.ds(base, CH)])

            return carry

        lax.fori_loop(0, per, body, 0)

    return k


def make_gather_emb(V, R):
    NCHK = R // CH
    per = -(-NCHK // NWK)

    @functools.partial(
        pl.kernel,
        out_type=jax.ShapeDtypeStruct((R, H), f32),
        mesh=_mesh(),
        scratch_types=[
            pltpu.VMEM((CH,), i32), pltpu.VMEM((CH, H), f32),
            pltpu.SemaphoreType.DMA,
        ],
    )
    def k(tab, idx, out, iv, rv, s0):
        wid = lax.axis_index("s") * NCORE + lax.axis_index("c")

        def body(j, carry):
            chk = j * NWK + wid

            @pl.when(chk < NCHK)
            def _():
                base = chk * CH
                pltpu.sync_copy(idx.at[pl.ds(base, CH)], iv)
                pltpu.async_copy(tab.at[iv], rv, s0).wait()
                pltpu.sync_copy(rv, out.at[pl.ds(base, CH)])

            return carry

        lax.fori_loop(0, per, body, 0)

    return k


def make_scatter(S, R, WS, NWIN):
    WSP = -(-(WS + 1) // 256) * 256
    SW = WSP // 16
    ZR = 16
    SOUT = NWIN * WS
    assert WS % 128 == 0 and SW % ZR == 0 and SOUT >= S

    @functools.partial(
        pl.kernel,
        out_type=jax.ShapeDtypeStruct((2, SOUT, H), f32),
        mesh=_mesh(),
        scratch_types=[
            pltpu.VMEM((CH,), i32),
            pltpu.VMEM((CH,), i32),
            pltpu.VMEM((CH, H), f32),
            pltpu.VMEM((ZR, H), f32),
            pltpu.VMEM((16,), i32),
            pltpu.VMEM_SHARED((WSP, H), f32),
        ],
    )
    def k(vals, keys, offs, out, kv, lv, rv, zb, ov, sh):
        cc = lax.axis_index("c")
        sid = lax.axis_index("s")

        def zb_body(t, carry):
            zb[t // 8, pl.ds((t % 8) * 16, 16)] = jnp.zeros((16,), f32)
            return carry

        lax.fori_loop(0, ZR * 8, zb_body, 0)

        ow = WS // 16

        def w_body(w, wcarry):
            def z_body(t, carry):
                off = pl.multiple_of(sid * SW + t * ZR, 8)
                pltpu.sync_copy(zb, sh.at[pl.ds(off, ZR)])
                return carry

            lax.fori_loop(0, SW // ZR, z_body, 0)
            plsc.subcore_barrier()

            pltpu.sync_copy(offs.at[w, sid], ov)
            ovv = ov[...]
            t0 = ovv[0]
            t1 = ovv[1]
            wbase = w * WS

            def c_body(ci, carry):
                base = ci * CH
                pltpu.sync_copy(keys.at[pl.ds(base, CH)], kv)
                for q in range(CH // 16):
                    kk = kv[pl.ds(q * 16, 16)]
                    li = kk - wbase
                    ok = (li >= 0) & (li < WS)
                    lv[pl.ds(q * 16, 16)] = jnp.where(ok, li, jnp.int32(WS))
                pltpu.sync_copy(vals.at[cc, pl.ds(base, CH)], rv)
                pltpu.sync_copy(rv, sh.at[lv], add=True)
                return carry

            lax.fori_loop(t0, t1, c_body, 0)
            plsc.subcore_barrier()

            so = pl.multiple_of(sid * ow, 8)
            do = pl.multiple_of(wbase + sid * ow, 8)
            pltpu.sync_copy(sh.at[pl.ds(so, ow)], out.at[cc, pl.ds(do, ow)])
            plsc.subcore_barrier()
            return wcarry

        lax.fori_loop(0, NWIN, w_body, 0)

    return k


def _full(shape):
    return pl.BlockSpec(shape, lambda i: tuple(0 for _ in shape))


def make_bonds(blk):
    grid = (E // blk,)
    step = 8.0 / (RBE - 1)
    gamma = 1.0 / step ** 2

    def body(r_ref, w1, b1, w2, b2, y_ref, rt_ref):
        r = r_ref[...]
        bl = jnp.sqrt(jnp.sum(r * r, axis=1, keepdims=True))
        centers = lax.broadcasted_iota(i32, (1, RBE), 1).astype(f32) * step
        rbf = jnp.exp(-gamma * (bl - centers) ** 2)
        h1 = _silu(_ln(jnp.dot(rbf, w1[...], precision=lax.Precision.HIGHEST, preferred_element_type=f32)
                       + b1[...]))
        y = _silu(_ln(jnp.dot(h1, w2[...], precision=lax.Precision.HIGHEST, preferred_element_type=f32)
                      + b2[...]))
        y_ref[...] = y
        fc2 = jnp.where(bl < CUT, 0.5 * (jnp.cos(jnp.pi * bl / CUT) + 1.0),
                        0.0)
        rt_ref[...] = jnp.concatenate(
            [r, fc2, jnp.zeros((blk, H - 4), f32)], axis=1)

    return pl.pallas_call(
        body,
        grid=grid,
        in_specs=[pl.BlockSpec((blk, 3), lambda i: (i, 0)),
                  _full((RBE, EMB)), _full((1, EMB)),
                  _full((EMB, H)), _full((1, H))],
        out_specs=[pl.BlockSpec((blk, H), lambda i: (i, 0)),
                   pl.BlockSpec((blk, H), lambda i: (i, 0))],
        out_shape=[jax.ShapeDtypeStruct((E, H), f32),
                   jax.ShapeDtypeStruct((E, H), f32)],
    )


def make_angles(blk):
    grid = (T // blk,)
    step = 2.0 / (RBA - 1)
    gamma = 1.0 / step ** 2

    def body(ra_ref, rb_ref, w1, b1, w2, b2, z_ref):
        ra = ra_ref[...]
        rb = rb_ref[...]
        r1 = -ra[:, 0:3]
        r2 = rb[:, 0:3]
        dot = jnp.sum(r1 * r2, axis=1, keepdims=True)
        n1 = jnp.sqrt(jnp.sum(r1 * r1, axis=1, keepdims=True))
        n2 = jnp.sqrt(jnp.sum(r2 * r2, axis=1, keepdims=True))
        cosang = jnp.clip(dot / (n1 * n2 + 1e-12), -1.0, 1.0)
        centers = (lax.broadcasted_iota(i32, (1, RBA), 1).astype(f32) * step
                   - 1.0)
        rbf = jnp.exp(-gamma * (cosang - centers) ** 2)
        h1 = _silu(_ln(jnp.dot(rbf, w1[...], precision=lax.Precision.HIGHEST, preferred_element_type=f32)
                       + b1[...]))
        z = _silu(_ln(jnp.dot(h1, w2[...], precision=lax.Precision.HIGHEST, preferred_element_type=f32)
                      + b2[...]))
        fcut3 = ra[:, 3:4] * rb[:, 3:4]
        z_ref[...] = z * fcut3

    return pl.pallas_call(
        body,
        grid=grid,
        in_specs=[pl.BlockSpec((blk, H), lambda i: (i, 0)),
                  pl.BlockSpec((blk, H), lambda i: (i, 0)),
                  _full((RBA, EMB)), _full((1, EMB)),
                  _full((EMB, H)), _full((1, H))],
        out_specs=pl.BlockSpec((blk, H), lambda i: (i, 0)),
        out_shape=jax.ShapeDtypeStruct((T, H), f32),
    )


def make_proj3(S, blk):
    grid = (S // blk,)

    def body(x_ref, w_ref, b_ref, a_ref, b2_ref, h_ref):
        x = x_ref[...]
        a_ref[...] = jnp.dot(x, w_ref[0], precision=lax.Precision.HIGHEST, preferred_element_type=f32) + b_ref[0]
        b2_ref[...] = jnp.dot(x, w_ref[1], precision=lax.Precision.HIGHEST, preferred_element_type=f32) + b_ref[1]
        h_ref[...] = jnp.dot(x, w_ref[2], precision=lax.Precision.HIGHEST, preferred_element_type=f32) + b_ref[2]

    return pl.pallas_call(
        body,
        grid=grid,
        in_specs=[pl.BlockSpec((blk, H), lambda i: (i, 0)),
                  pl.BlockSpec((3, H, H), lambda i: (0, 0, 0)),
                  pl.BlockSpec((3, 1, H), lambda i: (0, 0, 0))],
        out_specs=[pl.BlockSpec((blk, H), lambda i: (i, 0))] * 3,
        out_shape=[jax.ShapeDtypeStruct((S, H), f32)] * 3,
    )


def make_gate(R, blk, skip_edgenorm):
    grid = (R // blk,)

    def body(ga_ref, gb_ref, gh_ref, y_ref, w2, b2, nd_ref, yn_ref):
        y = y_ref[...]
        m = (ga_ref[...] + gb_ref[...]
             + jnp.dot(y, w2[...], precision=lax.Precision.HIGHEST, preferred_element_type=f32) + b2[...])
        sig = jax.nn.sigmoid(m)
        num = sig * gh_ref[...]
        nd_ref[...] = jnp.stack([num, sig], axis=0)
        ym = m if skip_edgenorm else _ln(m)
        yn_ref[...] = y + _silu(ym)

    return pl.pallas_call(
        body,
        grid=grid,
        in_specs=[pl.BlockSpec((blk, H), lambda i: (i, 0))] * 4
                 + [_full((H, H)), _full((1, H))],
        out_specs=[pl.BlockSpec((2, blk, H), lambda i: (0, i, 0)),
                   pl.BlockSpec((blk, H), lambda i: (i, 0))],
        out_shape=[jax.ShapeDtypeStruct((2, R, H), f32),
                   jax.ShapeDtypeStruct((R, H), f32)],
    )


def make_update(S, blk):
    grid = (S // blk,)

    def body(x_ref, nd_ref, w4, b4, o_ref):
        x = x_ref[...]
        h = nd_ref[0] / (nd_ref[1] + 1e-6)
        o_ref[...] = x + _silu(_ln(
            jnp.dot(x, w4[...], precision=lax.Precision.HIGHEST, preferred_element_type=f32) + b4[...] + h))

    return pl.pallas_call(
        body,
        grid=grid,
        in_specs=[pl.BlockSpec((blk, H), lambda i: (i, 0)),
                  pl.BlockSpec((2, blk, H), lambda i: (0, i, 0)),
                  _full((H, H)), _full((1, H))],
        out_specs=pl.BlockSpec((blk, H), lambda i: (i, 0)),
        out_shape=jax.ShapeDtypeStruct((S, H), f32),
    )


def make_readout():
    def body(x_ref, w_ref, b_ref, o_ref):
        s = jnp.sum(x_ref[...], axis=0, keepdims=True)
        o_ref[...] = (jnp.sum(s * w_ref[...], keepdims=True).reshape(1, 1)
                      + b_ref[...] * N)

    return pl.pallas_call(
        body,
        in_specs=[pl.BlockSpec((N, H), lambda: (0, 0)),
                  pl.BlockSpec((1, H), lambda: (0, 0)),
                  pl.BlockSpec((1, 1), lambda: (0, 0))],
        out_specs=pl.BlockSpec((1, 1), lambda: (0, 0)),
        out_shape=jax.ShapeDtypeStruct((1, 1), f32),
    )


WS_N = NP
WS_E = 12800
NWIN_E = -(-E // WS_E)


def _tile_offsets(woff):
    a = woff[:-1] // CH
    b = -(-woff[1:] // CH)
    sgrid = jnp.arange(17, dtype=i32)
    tt = a[:, None] + ((b - a)[:, None] * sgrid[None, :]) // 16
    packed = jnp.stack([tt[:, :16], tt[:, 1:17]], axis=2)
    return jnp.pad(packed, ((0, 0), (0, 0), (0, 14))).astype(i32)


def kernel(r, atomic_number, edge_index, lg_edge_index, atom_emb,
           edge_W1, edge_b1, edge_W2, edge_b2,
           angle_W1, angle_b1, angle_W2, angle_b2,
           alignn_W, alignn_b, gcn_W, gcn_b, fc_W, fc_b):
    edge_index = edge_index.astype(i32)
    lg = lg_edge_index.astype(i32)
    src = edge_index[0]

    perm = jnp.argsort(lg[0])
    lsrc = lg[0][perm]
    ldst = lg[1][perm]
    lidx2 = jnp.stack([lsrc, ldst])

    woff_n = jnp.array([0, E], dtype=i32)
    offs_n = _tile_offsets(woff_n)
    woff_e = jnp.searchsorted(lsrc, (jnp.arange(NWIN_E + 1) * WS_E)
                              .astype(i32)).astype(i32)
    offs_e = _tile_offsets(woff_e)

    g_emb = make_gather_emb(NZ, NP)
    g_abh_n = make_gather_abh(N, E)
    g_abh_e = make_gather_abh(E, T)
    g_rt = make_gather_pair(E, T, H)
    sc_n = make_scatter(NP, E, WS_N, 1)
    sc_e = make_scatter(E, T, WS_E, NWIN_E)

    bonds = make_bonds(2000)
    angles = make_angles(2000)
    proj3_n = make_proj3(N, 2000)
    proj3_e = make_proj3(E, 2000)
    gate_n = {s: make_gate(E, 2000, s) for s in (False, True)}
    gate_e = {s: make_gate(T, 2000, s) for s in (False, True)}
    upd_n = make_update(N, 2000)
    upd_e = make_update(E, 2000)
    readout = make_readout()

    an_pad = jnp.pad(atomic_number.astype(i32), (0, NP - N))
    x = g_emb(atom_emb, an_pad)[:N]

    y, rtab = bonds(r, edge_W1, edge_b1.reshape(1, EMB),
                    edge_W2, edge_b2.reshape(1, H))

    ra, rb = g_rt(rtab, lidx2)
    z = angles(ra, rb, angle_W1, angle_b1.reshape(1, EMB),
               angle_W2, angle_b2.reshape(1, H))

    def conv_n(x, y, W, b, skip):
        w013 = jnp.stack([W[0], W[1], W[3]])
        b013 = jnp.stack([b[0], b[1], b[3]]).reshape(3, 1, H)
        a, bb, hh = proj3_n(x, w013, b013)
        ga, gb, gh = g_abh_n(a, bb, hh, edge_index)
        nd, ynew = gate_n[skip](ga, gb, gh, y, W[2], b[2].reshape(1, H))
        acc = sc_n(nd, src, offs_n)[:, :N]
        xnew = upd_n(x, acc, W[4], b[4].reshape(1, H))
        return xnew, ynew

    def conv_e(x, y, W, b, skip):
        w013 = jnp.stack([W[0], W[1], W[3]])
        b013 = jnp.stack([b[0], b[1], b[3]]).reshape(3, 1, H)
        a, bb, hh = proj3_e(x, w013, b013)
        ga, gb, gh = g_abh_e(a, bb, hh, lidx2)
        nd, ynew = gate_e[skip](ga, gb, gh, y, W[2], b[2].reshape(1, H))
        acc = sc_e(nd, lsrc, offs_e)[:, :E]
        xnew = upd_e(x, acc, W[4], b[4].reshape(1, H))
        return xnew, ynew

    for i in range(NA):
        skip = (i == NA - 1)
        x, m = conv_n(x, y, alignn_W[i, 0], alignn_b[i, 0], False)
        y, z = conv_e(m, z, alignn_W[i, 1], alignn_b[i, 1], skip)
    for i in range(NG):
        skip = (i == NG - 1)
        x, y = conv_n(x, y, gcn_W[i], gcn_b[i], skip)

    energy = readout(x, fc_W.reshape(1, H), fc_b.reshape(1, 1))
    return jnp.reshape(energy, ())


# speedup vs baseline: 2.7619x; 1.0346x over previous
import functools

import jax
import jax.numpy as jnp
from jax import lax
from jax.experimental import pallas as pl
from jax.experimental.pallas import tpu as pltpu
from jax.experimental.pallas import tpu_sc as plsc

N = 10000
E = 160000
T = 320000
H = 128
EMB = 64
RBE = 80
RBA = 40
NZ = 108
NA = 4
NG = 4
CUT = 5.0

NCORE = 2
NSUB = 16
NWK = NCORE * NSUB
CH = 128
NP = 10112

f32 = jnp.float32
i32 = jnp.int32


def _mesh():
    return plsc.VectorSubcoreMesh(core_axis_name="c", subcore_axis_name="s",
                                  num_cores=NCORE, num_subcores=NSUB)


def _ln(h):
    mu = jnp.mean(h, axis=-1, keepdims=True)
    var = jnp.mean((h - mu) ** 2, axis=-1, keepdims=True)
    return (h - mu) / jnp.sqrt(var + 1e-5)


def _silu(x):
    return x * jax.nn.sigmoid(x)


def _bounds(wid, nchk):
    t0 = (wid * nchk) >> 5
    t1 = ((wid + 1) * nchk) >> 5
    return t0, t1


NB = 4
CHG = 64


def make_gather_abh(S, R):
    NCHK = R // CHG
    LMAX = -(-NCHK // NWK)
    LMAXE = -(-LMAX // 2) * 2
    NIT = -(-LMAX // NB)

    @functools.partial(
        pl.kernel,
        out_type=[jax.ShapeDtypeStruct((R, H), f32)] * 3,
        mesh=_mesh(),
        scratch_types=[
            pltpu.VMEM((LMAXE * CHG,), i32), pltpu.VMEM((LMAXE * CHG,), i32),
            pltpu.VMEM((NB, CHG, H), f32), pltpu.VMEM((NB, CHG, H), f32),
            pltpu.VMEM((NB, CHG, H), f32),
        ] + [pltpu.SemaphoreType.DMA] * (2 * NB),
    )
    def k(ta, tb, th, srci, dsti, oa, ob, oh, ivs, ivd, ra, rb, rh, *sems):
        gs = sems[:NB]
        ws = sems[NB:]
        wid = lax.axis_index("s") * NCORE + lax.axis_index("c")
        t0, t1 = _bounds(wid, NCHK)
        nloc = t1 - t0
        t0c = jnp.minimum(t0, NCHK - LMAXE)
        sh_ = t0 - t0c
        pltpu.sync_copy(srci.at[pl.ds(t0c * CHG, LMAXE * CHG)], ivs)
        pltpu.sync_copy(dsti.at[pl.ds(t0c * CHG, LMAXE * CHG)], ivd)

        def gath(b, c):
            s_ = ivs.at[pl.ds((sh_ + c) * CHG, CHG)]
            d_ = ivd.at[pl.ds((sh_ + c) * CHG, CHG)]
            return [pltpu.make_async_copy(ta.at[s_], ra.at[b], gs[b]),
                    pltpu.make_async_copy(tb.at[d_], rb.at[b], gs[b]),
                    pltpu.make_async_copy(th.at[d_], rh.at[b], gs[b])]

        def wrts(b, base):
            return [pltpu.make_async_copy(ra.at[b], oa.at[pl.ds(base, CHG)],
                                          ws[b]),
                    pltpu.make_async_copy(rb.at[b], ob.at[pl.ds(base, CHG)],
                                          ws[b]),
                    pltpu.make_async_copy(rh.at[b], oh.at[pl.ds(base, CHG)],
                                          ws[b])]

        def body(j, carry):
            for b in range(NB):
                c = j * NB + b

                @pl.when((c < nloc) & (j > 0))
                def _():
                    for d in wrts(b, (t0 + c - NB) * CHG):
                        d.wait()

                @pl.when(c < nloc)
                def _():
                    for d in gath(b, c):
                        d.start()

            for b in range(NB):
                c = j * NB + b

                @pl.when(c < nloc)
                def _():
                    for d in gath(b, c):
                        d.wait()
                    for d in wrts(b, (t0 + c) * CHG):
                        d.start()

            return carry

        lax.fori_loop(0, NIT, body, 0)
        for b in range(NB):
            @pl.when(nloc > b)
            def _():
                for d in wrts(b, t0 * CHG):
                    d.wait()

    return k


def make_gather_pair(V, R, D):
    NCHK = R // CHG
    LMAX = -(-NCHK // NWK)
    LMAXE = -(-LMAX // 2) * 2
    NIT = -(-LMAX // NB)

    @functools.partial(
        pl.kernel,
        out_type=[jax.ShapeDtypeStruct((R, D), f32)] * 2,
        mesh=_mesh(),
        scratch_types=[
            pltpu.VMEM((LMAXE * CHG,), i32), pltpu.VMEM((LMAXE * CHG,), i32),
            pltpu.VMEM((NB, CHG, D), f32), pltpu.VMEM((NB, CHG, D), f32),
        ] + [pltpu.SemaphoreType.DMA] * (2 * NB),
    )
    def k(tab, srci, dsti, oa, ob, ivs, ivd, ra, rb, *sems):
        gs = sems[:NB]
        ws = sems[NB:]
        wid = lax.axis_index("s") * NCORE + lax.axis_index("c")
        t0, t1 = _bounds(wid, NCHK)
        nloc = t1 - t0
        t0c = jnp.minimum(t0, NCHK - LMAXE)
        sh_ = t0 - t0c
        pltpu.sync_copy(srci.at[pl.ds(t0c * CHG, LMAXE * CHG)], ivs)
        pltpu.sync_copy(dsti.at[pl.ds(t0c * CHG, LMAXE * CHG)], ivd)

        def gath(b, c):
            s_ = ivs.at[pl.ds((sh_ + c) * CHG, CHG)]
            d_ = ivd.at[pl.ds((sh_ + c) * CHG, CHG)]
            return [pltpu.make_async_copy(tab.at[s_], ra.at[b], gs[b]),
                    pltpu.make_async_copy(tab.at[d_], rb.at[b], gs[b])]

        def wrts(b, base):
            return [pltpu.make_async_copy(ra.at[b], oa.at[pl.ds(base, CHG)],
                                          ws[b]),
                    pltpu.make_async_copy(rb.at[b], ob.at[pl.ds(base, CHG)],
                                          ws[b])]

        def body(j, carry):
            for b in range(NB):
                c = j * NB + b

                @pl.when((c < nloc) & (j > 0))
                def _():
                    for d in wrts(b, (t0 + c - NB) * CHG):
                        d.wait()

                @pl.when(c < nloc)
                def _():
                    for d in gath(b, c):
                        d.start()

            for b in range(NB):
                c = j * NB + b

                @pl.when(c < nloc)
                def _():
                    for d in gath(b, c):
                        d.wait()
                    for d in wrts(b, (t0 + c) * CHG):
                        d.start()

            return carry

        lax.fori_loop(0, NIT, body, 0)
        for b in range(NB):
            @pl.when(nloc > b)
            def _():
                for d in wrts(b, t0 * CHG):
                    d.wait()

    return k


def make_gather_emb(V, R):
    NCHK = R // CHG
    LMAX = -(-NCHK // NWK)
    LMAXE = -(-LMAX // 2) * 2
    NIT = -(-LMAX // NB)

    @functools.partial(
        pl.kernel,
        out_type=jax.ShapeDtypeStruct((R, H), f32),
        mesh=_mesh(),
        scratch_types=[
            pltpu.VMEM((LMAXE * CHG,), i32),
            pltpu.VMEM((NB, CHG, H), f32),
        ] + [pltpu.SemaphoreType.DMA] * (2 * NB),
    )
    def k(tab, idx, out, iv, rv, *sems):
        gs = sems[:NB]
        ws = sems[NB:]
        wid = lax.axis_index("s") * NCORE + lax.axis_index("c")
        t0, t1 = _bounds(wid, NCHK)
        nloc = t1 - t0
        t0c = jnp.minimum(t0, NCHK - LMAXE)
        sh_ = t0 - t0c
        pltpu.sync_copy(idx.at[pl.ds(t0c * CHG, LMAXE * CHG)], iv)

        def gath(b, c):
            s_ = iv.at[pl.ds((sh_ + c) * CHG, CHG)]
            return pltpu.make_async_copy(tab.at[s_], rv.at[b], gs[b])

        def wrt(b, base):
            return pltpu.make_async_copy(rv.at[b], out.at[pl.ds(base, CHG)],
                                         ws[b])

        def body(j, carry):
            for b in range(NB):
                c = j * NB + b

                @pl.when((c < nloc) & (j > 0))
                def _():
                    wrt(b, (t0 + c - NB) * CHG).wait()

                @pl.when(c < nloc)
                def _():
                    gath(b, c).start()

            for b in range(NB):
                c = j * NB + b

                @pl.when(c < nloc)
                def _():
                    gath(b, c).wait()
                    wrt(b, (t0 + c) * CHG).start()

            return carry

        lax.fori_loop(0, NIT, body, 0)
        for b in range(NB):
            @pl.when(nloc > b)
            def _():
                wrt(b, t0 * CHG).wait()

    return k


def make_scatter(S, R, WS, NWIN):
    WSP = -(-(WS + 1) // 256) * 256
    SW = WSP // 16
    ZR = 16
    SOUT = NWIN * WS
    assert WS % 128 == 0 and SW % ZR == 0 and SOUT >= S

    @functools.partial(
        pl.kernel,
        out_type=jax.ShapeDtypeStruct((2, SOUT, H), f32),
        mesh=_mesh(),
        scratch_types=[
            pltpu.VMEM((CH,), i32),
            pltpu.VMEM((CH,), i32),
            pltpu.VMEM((CH, H), f32),
            pltpu.VMEM((ZR, H), f32),
            pltpu.VMEM((16,), i32),
            pltpu.VMEM_SHARED((WSP, H), f32),
        ],
    )
    def k(vals, keys, offs, out, kv, lv, rv, zb, ov, sh):
        cc = lax.axis_index("c")
        sid = lax.axis_index("s")

        def zb_body(t, carry):
            zb[t // 8, pl.ds((t % 8) * 16, 16)] = jnp.zeros((16,), f32)
            return carry

        lax.fori_loop(0, ZR * 8, zb_body, 0)

        ow = WS // 16

        def w_body(w, wcarry):
            def z_body(t, carry):
                off = pl.multiple_of(sid * SW + t * ZR, 8)
                pltpu.sync_copy(zb, sh.at[pl.ds(off, ZR)])
                return carry

            lax.fori_loop(0, SW // ZR, z_body, 0)
            plsc.subcore_barrier()

            pltpu.sync_copy(offs.at[w, sid], ov)
            ovv = ov[...]
            t0 = ovv[0]
            t1 = ovv[1]
            wbase = w * WS

            def c_body(ci, carry):
                base = ci * CH
                pltpu.sync_copy(keys.at[pl.ds(base, CH)], kv)
                for q in range(CH // 16):
                    kk = kv[pl.ds(q * 16, 16)]
                    li = kk - wbase
                    ok = (li >= 0) & (li < WS)
                    lv[pl.ds(q * 16, 16)] = jnp.where(ok, li, jnp.int32(WS))
                pltpu.sync_copy(vals.at[cc, pl.ds(base, CH)], rv)
                pltpu.sync_copy(rv, sh.at[lv], add=True)
                return carry

            lax.fori_loop(t0, t1, c_body, 0)
            plsc.subcore_barrier()

            so = pl.multiple_of(sid * ow, 8)
            do = pl.multiple_of(wbase + sid * ow, 8)
            pltpu.sync_copy(sh.at[pl.ds(so, ow)], out.at[cc, pl.ds(do, ow)])
            plsc.subcore_barrier()
            return wcarry

        lax.fori_loop(0, NWIN, w_body, 0)

    return k


def _full(shape):
    return pl.BlockSpec(shape, lambda i: tuple(0 for _ in shape))


def make_bonds(blk):
    grid = (E // blk,)
    step = 8.0 / (RBE - 1)
    gamma = 1.0 / step ** 2

    def body(r_ref, w1, b1, w2, b2, y_ref, rt_ref):
        r = r_ref[...]
        bl = jnp.sqrt(jnp.sum(r * r, axis=1, keepdims=True))
        centers = lax.broadcasted_iota(i32, (1, RBE), 1).astype(f32) * step
        rbf = jnp.exp(-gamma * (bl - centers) ** 2)
        h1 = _silu(_ln(jnp.dot(rbf, w1[...], precision=lax.Precision.HIGHEST, preferred_element_type=f32)
                       + b1[...]))
        y = _silu(_ln(jnp.dot(h1, w2[...], precision=lax.Precision.HIGHEST, preferred_element_type=f32)
                      + b2[...]))
        y_ref[...] = y
        fc2 = jnp.where(bl < CUT, 0.5 * (jnp.cos(jnp.pi * bl / CUT) + 1.0),
                        0.0)
        rt_ref[...] = jnp.concatenate(
            [r, fc2, jnp.zeros((blk, H - 4), f32)], axis=1)

    return pl.pallas_call(
        body,
        grid=grid,
        in_specs=[pl.BlockSpec((blk, 3), lambda i: (i, 0)),
                  _full((RBE, EMB)), _full((1, EMB)),
                  _full((EMB, H)), _full((1, H))],
        out_specs=[pl.BlockSpec((blk, H), lambda i: (i, 0)),
                   pl.BlockSpec((blk, H), lambda i: (i, 0))],
        out_shape=[jax.ShapeDtypeStruct((E, H), f32),
                   jax.ShapeDtypeStruct((E, H), f32)],
    )


def make_angles(blk):
    grid = (T // blk,)
    step = 2.0 / (RBA - 1)
    gamma = 1.0 / step ** 2

    def body(ra_ref, rb_ref, w1, b1, w2, b2, z_ref):
        ra = ra_ref[...]
        rb = rb_ref[...]
        r1 = -ra[:, 0:3]
        r2 = rb[:, 0:3]
        dot = jnp.sum(r1 * r2, axis=1, keepdims=True)
        n1 = jnp.sqrt(jnp.sum(r1 * r1, axis=1, keepdims=True))
        n2 = jnp.sqrt(jnp.sum(r2 * r2, axis=1, keepdims=True))
        cosang = jnp.clip(dot / (n1 * n2 + 1e-12), -1.0, 1.0)
        centers = (lax.broadcasted_iota(i32, (1, RBA), 1).astype(f32) * step
                   - 1.0)
        rbf = jnp.exp(-gamma * (cosang - centers) ** 2)
        h1 = _silu(_ln(jnp.dot(rbf, w1[...], precision=lax.Precision.HIGHEST, preferred_element_type=f32)
                       + b1[...]))
        z = _silu(_ln(jnp.dot(h1, w2[...], precision=lax.Precision.HIGHEST, preferred_element_type=f32)
                      + b2[...]))
        fcut3 = ra[:, 3:4] * rb[:, 3:4]
        z_ref[...] = z * fcut3

    return pl.pallas_call(
        body,
        grid=grid,
        in_specs=[pl.BlockSpec((blk, H), lambda i: (i, 0)),
                  pl.BlockSpec((blk, H), lambda i: (i, 0)),
                  _full((RBA, EMB)), _full((1, EMB)),
                  _full((EMB, H)), _full((1, H))],
        out_specs=pl.BlockSpec((blk, H), lambda i: (i, 0)),
        out_shape=jax.ShapeDtypeStruct((T, H), f32),
    )


def make_proj3(S, blk):
    grid = (S // blk,)

    def body(x_ref, w_ref, b_ref, a_ref, b2_ref, h_ref):
        x = x_ref[...]
        a_ref[...] = jnp.dot(x, w_ref[0], precision=lax.Precision.HIGHEST, preferred_element_type=f32) + b_ref[0]
        b2_ref[...] = jnp.dot(x, w_ref[1], precision=lax.Precision.HIGHEST, preferred_element_type=f32) + b_ref[1]
        h_ref[...] = jnp.dot(x, w_ref[2], precision=lax.Precision.HIGHEST, preferred_element_type=f32) + b_ref[2]

    return pl.pallas_call(
        body,
        grid=grid,
        in_specs=[pl.BlockSpec((blk, H), lambda i: (i, 0)),
                  pl.BlockSpec((3, H, H), lambda i: (0, 0, 0)),
                  pl.BlockSpec((3, 1, H), lambda i: (0, 0, 0))],
        out_specs=[pl.BlockSpec((blk, H), lambda i: (i, 0))] * 3,
        out_shape=[jax.ShapeDtypeStruct((S, H), f32)] * 3,
    )


def make_gate(R, blk, skip_edgenorm):
    grid = (R // blk,)

    def body(ga_ref, gb_ref, gh_ref, y_ref, w2, b2, nd_ref, yn_ref):
        y = y_ref[...]
        m = (ga_ref[...] + gb_ref[...]
             + jnp.dot(y, w2[...], precision=lax.Precision.HIGHEST, preferred_element_type=f32) + b2[...])
        sig = jax.nn.sigmoid(m)
        num = sig * gh_ref[...]
        nd_ref[...] = jnp.stack([num, sig], axis=0)
        ym = m if skip_edgenorm else _ln(m)
        yn_ref[...] = y + _silu(ym)

    return pl.pallas_call(
        body,
        grid=grid,
        in_specs=[pl.BlockSpec((blk, H), lambda i: (i, 0))] * 4
                 + [_full((H, H)), _full((1, H))],
        out_specs=[pl.BlockSpec((2, blk, H), lambda i: (0, i, 0)),
                   pl.BlockSpec((blk, H), lambda i: (i, 0))],
        out_shape=[jax.ShapeDtypeStruct((2, R, H), f32),
                   jax.ShapeDtypeStruct((R, H), f32)],
    )


def make_update(S, blk):
    grid = (S // blk,)

    def body(x_ref, nd_ref, w4, b4, o_ref):
        x = x_ref[...]
        h = nd_ref[0] / (nd_ref[1] + 1e-6)
        o_ref[...] = x + _silu(_ln(
            jnp.dot(x, w4[...], precision=lax.Precision.HIGHEST, preferred_element_type=f32) + b4[...] + h))

    return pl.pallas_call(
        body,
        grid=grid,
        in_specs=[pl.BlockSpec((blk, H), lambda i: (i, 0)),
                  pl.BlockSpec((2, blk, H), lambda i: (0, i, 0)),
                  _full((H, H)), _full((1, H))],
        out_specs=pl.BlockSpec((blk, H), lambda i: (i, 0)),
        out_shape=jax.ShapeDtypeStruct((S, H), f32),
    )


def make_readout():
    def body(x_ref, w_ref, b_ref, o_ref):
        s = jnp.sum(x_ref[...], axis=0, keepdims=True)
        o_ref[...] = (jnp.sum(s * w_ref[...], keepdims=True).reshape(1, 1)
                      + b_ref[...] * N)

    return pl.pallas_call(
        body,
        in_specs=[pl.BlockSpec((N, H), lambda: (0, 0)),
                  pl.BlockSpec((1, H), lambda: (0, 0)),
                  pl.BlockSpec((1, 1), lambda: (0, 0))],
        out_specs=pl.BlockSpec((1, 1), lambda: (0, 0)),
        out_shape=jax.ShapeDtypeStruct((1, 1), f32),
    )


WS_N = NP
WS_E = 12800
NWIN_E = -(-E // WS_E)


def _tile_offsets(woff):
    a = woff[:-1] // CH
    b = -(-woff[1:] // CH)
    sgrid = jnp.arange(17, dtype=i32)
    tt = a[:, None] + ((b - a)[:, None] * sgrid[None, :]) // 16
    packed = jnp.stack([tt[:, :16], tt[:, 1:17]], axis=2)
    return jnp.pad(packed, ((0, 0), (0, 0), (0, 14))).astype(i32)


def kernel(r, atomic_number, edge_index, lg_edge_index, atom_emb,
           edge_W1, edge_b1, edge_W2, edge_b2,
           angle_W1, angle_b1, angle_W2, angle_b2,
           alignn_W, alignn_b, gcn_W, gcn_b, fc_W, fc_b):
    edge_index = edge_index.astype(i32)
    lg = lg_edge_index.astype(i32)
    src = edge_index[0]
    dst = edge_index[1]

    perm = jnp.argsort(lg[0])
    lsrc = lg[0][perm]
    ldst = lg[1][perm]
    lidx2 = jnp.stack([lsrc, ldst])

    woff_n = jnp.array([0, E], dtype=i32)
    offs_n = _tile_offsets(woff_n)
    woff_e = jnp.searchsorted(lsrc, (jnp.arange(NWIN_E + 1) * WS_E)
                              .astype(i32)).astype(i32)
    offs_e = _tile_offsets(woff_e)

    g_emb = make_gather_emb(NZ, NP)
    g_abh_n = make_gather_abh(N, E)
    g_abh_e = make_gather_abh(E, T)
    g_rt = make_gather_pair(E, T, H)
    sc_n = make_scatter(NP, E, WS_N, 1)
    sc_e = make_scatter(E, T, WS_E, NWIN_E)

    bonds = make_bonds(2000)
    angles = make_angles(2000)
    proj3_n = make_proj3(N, 2000)
    proj3_e = make_proj3(E, 2000)
    gate_n = {s: make_gate(E, 2000, s) for s in (False, True)}
    gate_e = {s: make_gate(T, 2000, s) for s in (False, True)}
    upd_n = make_update(N, 2000)
    upd_e = make_update(E, 2000)
    readout = make_readout()

    an_pad = jnp.pad(atomic_number.astype(i32), (0, NP - N))
    x = g_emb(atom_emb, an_pad)[:N]

    y, rtab = bonds(r, edge_W1, edge_b1.reshape(1, EMB),
                    edge_W2, edge_b2.reshape(1, H))

    ra, rb = g_rt(rtab, lsrc, ldst)
    z = angles(ra, rb, angle_W1, angle_b1.reshape(1, EMB),
               angle_W2, angle_b2.reshape(1, H))

    def conv_n(x, y, W, b, skip):
        w013 = jnp.stack([W[0], W[1], W[3]])
        b013 = jnp.stack([b[0], b[1], b[3]]).reshape(3, 1, H)
        a, bb, hh = proj3_n(x, w013, b013)
        ga, gb, gh = g_abh_n(a, bb, hh, src, dst)
        nd, ynew = gate_n[skip](ga, gb, gh, y, W[2], b[2].reshape(1, H))
        acc = sc_n(nd, src, offs_n)[:, :N]
        xnew = upd_n(x, acc, W[4], b[4].reshape(1, H))
        return xnew, ynew

    def conv_e(x, y, W, b, skip):
        w013 = jnp.stack([W[0], W[1], W[3]])
        b013 = jnp.stack([b[0], b[1], b[3]]).reshape(3, 1, H)
        a, bb, hh = proj3_e(x, w013, b013)
        ga, gb, gh = g_abh_e(a, bb, hh, lsrc, ldst)
        nd, ynew = gate_e[skip](ga, gb, gh, y, W[2], b[2].reshape(1, H))
        acc = sc_e(nd, lsrc, offs_e)[:, :E]
        xnew = upd_e(x, acc, W[4], b[4].reshape(1, H))
        return xnew, ynew

    for i in range(NA):
        skip = (i == NA - 1)
        x, m = conv_n(x, y, alignn_W[i, 0], alignn_b[i, 0], False)
        y, z = conv_e(m, z, alignn_W[i, 1], alignn_b[i, 1], skip)
    for i in range(NG):
        skip = (i == NG - 1)
        x, y = conv_n(x, y, gcn_W[i], gcn_b[i], skip)

    energy = readout(x, fc_W.reshape(1, H), fc_b.reshape(1, 1))
    return jnp.reshape(energy, ())
